# Initial kernel scaffold; baseline (speedup 1.0000x reference)
import jax
import jax.numpy as jnp
from jax.experimental import pallas as pl


def kernel(x):
    raise NotImplementedError("write your pallas kernel here")


# speedup vs baseline: 347.8393x; 347.8393x over previous
import functools

import jax
import jax.numpy as jnp
from jax import lax
from jax.experimental import pallas as pl
from jax.experimental.pallas import tpu as pltpu
from jax.experimental.pallas import tpu_sc as plsc

_SCALE_H = 2.05
_SCALE_W = 3.15
_H = 512
_W = 512
_OH = int(round(_H * _SCALE_H))
_OW = int(round(_W * _SCALE_W))
_WPAD = 1616
_NC = 2
_NS = 16
_NW = _NC * _NS
_SRC_PER_W = _H // _NW
_RCH = 33
_RPAD = _NW * _RCH
_CCHUNKS = _WPAD // 16


def _trace_indices():
    oy = jnp.arange(_OH)
    ox = jnp.arange(_OW)
    iy1 = jnp.floor(oy.astype(jnp.float32) / _SCALE_H).astype(jnp.int32)
    ix1 = jnp.floor(ox.astype(jnp.float32) / _SCALE_W).astype(jnp.int32)
    iy1 = jnp.clip(iy1, 0, _H - 1)
    ix1 = jnp.clip(ix1, 0, _W - 1)
    fy = (oy.astype(jnp.float32) + 0.5) / _SCALE_H - 0.5
    fx = (ox.astype(jnp.float32) + 0.5) / _SCALE_W - 0.5
    iy2 = jnp.clip(jnp.round(fy).astype(jnp.int32), 0, _H - 1)
    ix2 = jnp.clip(jnp.round(fx).astype(jnp.int32), 0, _W - 1)
    ix1p = jnp.pad(ix1, (0, _WPAD - _OW), mode="edge")
    ix2p = jnp.pad(ix2, (0, _WPAD - _OW), mode="edge")
    iy1p = jnp.pad(iy1, (0, _RPAD - _OH)).reshape(_NW, _RCH)
    iy2p = jnp.pad(iy2, (0, _RPAD - _OH)).reshape(_NW, _RCH)
    return ix1p, ix2p, iy1p, iy2p

@functools.lru_cache(maxsize=1)
def _build():
    mesh = plsc.VectorSubcoreMesh(
        core_axis_name="c", subcore_axis_name="s", num_cores=_NC, num_subcores=_NS
    )

    @functools.partial(
        pl.kernel,
        out_type=(
            jax.ShapeDtypeStruct((_H, _WPAD), jnp.float32),
            jax.ShapeDtypeStruct((_H, _WPAD), jnp.float32),
        ),
        mesh=mesh,
        compiler_params=pltpu.CompilerParams(
            use_tc_tiling_on_sc=False, needs_layout_passes=False
        ),
        scratch_types=[
            pltpu.VMEM((_SRC_PER_W, _W), jnp.float32),
            pltpu.VMEM((_WPAD,), jnp.int32),
            pltpu.VMEM((_WPAD,), jnp.int32),
            pltpu.VMEM((_SRC_PER_W, _WPAD), jnp.float32),
            pltpu.VMEM((_SRC_PER_W, _WPAD), jnp.float32),
        ],
    )
    def col_gather(x_hbm, cx1_hbm, cx2_hbm, g1_hbm, g2_hbm, xv, cx1v, cx2v, g1v, g2v):
        wid = lax.axis_index("s") * _NC + lax.axis_index("c")
        base = wid * _SRC_PER_W
        pltpu.sync_copy(x_hbm.at[pl.ds(base, _SRC_PER_W), :], xv)
        pltpu.sync_copy(cx1_hbm, cx1v)
        pltpu.sync_copy(cx2_hbm, cx2v)

        def cbody(c, carry):
            colv1 = cx1v[pl.ds(c * 16, 16)]
            colv2 = cx2v[pl.ds(c * 16, 16)]
            for i in range(_SRC_PER_W):
                rowv = jnp.full((16,), i, jnp.int32)
                g1v[i, pl.ds(c * 16, 16)] = plsc.load_gather(xv, [rowv, colv1])
                g2v[i, pl.ds(c * 16, 16)] = plsc.load_gather(xv, [rowv, colv2])
            return carry

        lax.fori_loop(0, _CCHUNKS, cbody, 0)
        pltpu.sync_copy(g1v, g1_hbm.at[pl.ds(base, _SRC_PER_W), :])
        pltpu.sync_copy(g2v, g2_hbm.at[pl.ds(base, _SRC_PER_W), :])

    @functools.partial(
        pl.kernel,
        out_type=(
            jax.ShapeDtypeStruct((_RPAD, _WPAD), jnp.float32),
            jax.ShapeDtypeStruct((_RPAD, _WPAD), jnp.float32),
        ),
        mesh=mesh,
        compiler_params=pltpu.CompilerParams(
            use_tc_tiling_on_sc=False, needs_layout_passes=False
        ),
        scratch_types=[
            pltpu.VMEM((_RCH,), jnp.int32),
            pltpu.VMEM((_RCH, _WPAD), jnp.float32),
            pltpu.SemaphoreType.DMA,
        ],
    )
    def row_gather(g1_hbm, g2_hbm, iy1_hbm, iy2_hbm, o1_hbm, o2_hbm, idxv, rowsv, sem):
        wid = lax.axis_index("s") * _NC + lax.axis_index("c")
        base = wid * _RCH
        pltpu.sync_copy(iy1_hbm.at[wid], idxv)
        pltpu.async_copy(g1_hbm.at[idxv], rowsv, sem).wait()
        pltpu.sync_copy(rowsv, o1_hbm.at[pl.ds(base, _RCH), :])
        pltpu.sync_copy(iy2_hbm.at[wid], idxv)
        pltpu.async_copy(g2_hbm.at[idxv], rowsv, sem).wait()
        pltpu.sync_copy(rowsv, o2_hbm.at[pl.ds(base, _RCH), :])

    return col_gather, row_gather


def kernel(x):
    x2d = x[0, 0]
    cx1, cx2, iy1, iy2 = _trace_indices()
    col_gather, row_gather = _build()
    g1, g2 = col_gather(x2d, cx1, cx2)
    o1p, o2p = row_gather(g1, g2, iy1, iy2)
    out1 = o1p[:_OH, :_OW][None, None]
    out2 = o2p[:_OH, :_OW][None, None]
    return (out1, out2)
